# baseline (speedup 1.0000x reference)
import functools

import jax
import jax.numpy as jnp
from jax import lax
from jax.experimental import pallas as pl
from jax.experimental.pallas import tpu as pltpu
from jax.experimental.pallas import tpu_sc as plsc

D = 32
N_ROWS = 1000000
N_NEI = 16384
N_PAIRS = 65536
N_TRAIN = 65536
SCALE = float(2.0 ** -16)

NC = 2
NS = 16
NW = NC * NS
L = 16

MAP_SIZE = 1 << 20
MROWS_PER_TILE = MAP_SIZE // NS
N_PROWS = 2 * N_NEI
PAIRS_PER_W = N_PAIRS // NW
WAVE = 8

_mesh = plsc.VectorSubcoreMesh(core_axis_name="c", subcore_axis_name="s")


@functools.partial(
    pl.kernel,
    out_type=(
        jax.ShapeDtypeStruct((N_PAIRS,), jnp.float32),
        jax.ShapeDtypeStruct((2 * NC * MAP_SIZE,), jnp.float32),
    ),
    mesh=_mesh,
    scratch_types=[
        pltpu.VMEM((MROWS_PER_TILE,), jnp.float32),
        pltpu.VMEM((N_NEI,), jnp.int32),
        pltpu.VMEM((L,), jnp.int32),
        pltpu.VMEM((PAIRS_PER_W,), jnp.int32),
        pltpu.VMEM((PAIRS_PER_W,), jnp.int32),
        pltpu.VMEM((PAIRS_PER_W,), jnp.int32),
        pltpu.VMEM((PAIRS_PER_W,), jnp.int32),
        pltpu.VMEM((PAIRS_PER_W,), jnp.float32),
        pltpu.VMEM((PAIRS_PER_W,), jnp.float32),
        pltpu.VMEM((PAIRS_PER_W,), jnp.float32),
        pltpu.VMEM((PAIRS_PER_W,), jnp.float32),
        pltpu.VMEM((PAIRS_PER_W,), jnp.float32),
        pltpu.VMEM((PAIRS_PER_W,), jnp.float32),
        pltpu.VMEM((PAIRS_PER_W,), jnp.float32),
        pltpu.VMEM((PAIRS_PER_W,), jnp.float32),
        pltpu.VMEM((PAIRS_PER_W,), jnp.float32),
        pltpu.SemaphoreType.DMA,
        pltpu.SemaphoreType.DMA,
        pltpu.SemaphoreType.DMA,
        pltpu.SemaphoreType.DMA,
    ],
    compiler_params=pltpu.CompilerParams(
        needs_layout_passes=False, use_tc_tiling_on_sc=False),
)
def _score_all(ut_t, it_t, p_t, nei_u_hbm, nei_i_hbm, tu_hbm, ti_hbm,
               scores_ref, maps_ref,
               BUF, nei, tmp16, tu_v, ti_v, cu_v, ci_v,
               su_v, si_v, ei0_v, ei1_v, du0_v, du1_v, di0_v, di1_v, acc_v,
               semU, semG0, semG1, semM):
    c = lax.axis_index("c")
    s = lax.axis_index("s")
    w = s * NC + c
    pair_base = w * PAIRS_PER_W
    lanes = lax.iota(jnp.int32, L)
    shift_idx = jnp.minimum(lanes + 1, L - 1)
    is_last_lane = lanes == (L - 1)
    scale_f = jnp.float32(SCALE)
    zero_f = jnp.float32(0.0)
    NV = PAIRS_PER_W // L

    pltpu.sync_copy(tu_hbm.at[pl.ds(pair_base, PAIRS_PER_W)], tu_v)
    pltpu.sync_copy(ti_hbm.at[pl.ds(pair_base, PAIRS_PER_W)], ti_v)

    map_row_base = s * MROWS_PER_TILE
    neg1 = jnp.full((L,), -1.0, jnp.float32)

    def build_map(nei_hbm, which):
        pltpu.sync_copy(nei_hbm, nei)

        def init_body(i, carry):
            BUF2[pl.ds(i * L, L)] = neg1
            return carry

        def scan_body(g, carry):
            idx = nei[pl.ds(g * L, L)]
            key = (idx << 4) | lanes
            kvec = g * L + lanes
            skey, sval = plsc.sort_key_val(key, kvec)
            sidx = skey >> 4
            tmp16[...] = sidx
            nxt = plsc.load_gather(tmp16, [shift_idx])
            winner = (sidx != nxt) | is_last_lane
            local = sidx - map_row_base
            in_rng = (plsc.bitcast(local, jnp.uint32)
                      < jnp.uint32(MROWS_PER_TILE))
            local_c = local & (MROWS_PER_TILE - 1)
            plsc.store_scatter(BUF2, [local_c], sval.astype(jnp.float32),
                               mask=winner & in_rng)
            return carry

        lax.fori_loop(0, MROWS_PER_TILE // L, init_body, 0, unroll=8)
        lax.fori_loop(0, N_NEI // L, scan_body, 0)
        off = (c * 2 + which) * MAP_SIZE + map_row_base
        pltpu.sync_copy(BUF2, maps_ref.at[pl.ds(off, MROWS_PER_TILE)])

    BUF2 = BUF
    build_map(nei_u_hbm, 0)
    build_map(nei_i_hbm, 1)
    plsc.subcore_barrier()

    def u_start(d):
        pltpu.async_copy(ut_t.at[d].at[tu_v],
                         BUF.at[pl.ds(d * PAIRS_PER_W, PAIRS_PER_W)], semU)

    def u_drain(d):
        pltpu.make_async_copy(
            ut_t.at[0].at[tu_v],
            BUF.at[pl.ds(d * PAIRS_PER_W, PAIRS_PER_W)], semU).wait()

    for dd in range(0, D, WAVE):
        for d in range(dd, dd + WAVE):
            u_start(d)
        for d in range(dd, dd + WAVE):
            u_drain(d)


    def midx_body(v, carry):
        r = pl.ds(v * L, L)
        cu_v[r] = tu_v[r] + (c * 2) * MAP_SIZE
        ci_v[r] = ti_v[r] + (c * 2 + 1) * MAP_SIZE
        return carry

    lax.fori_loop(0, NV, midx_body, 0, unroll=4)
    cp1 = pltpu.async_copy(maps_ref.at[cu_v], du0_v, semM)
    cp2 = pltpu.async_copy(maps_ref.at[ci_v], di0_v, semM)
    cp1.wait()
    cp2.wait()

    def scl_body(v, carry):
        r = pl.ds(v * L, L)
        spread = (pair_base + v * L + lanes) & (N_NEI - 1)
        sf_u = du0_v[r]
        mu = sf_u >= zero_f
        su_v[r] = jnp.where(mu, scale_f, zero_f)
        cu_v[r] = jnp.where(mu, sf_u.astype(jnp.int32), spread)
        sf_i = di0_v[r]
        mi = sf_i >= zero_f
        si_v[r] = jnp.where(mi, scale_f, zero_f)
        ci_v[r] = jnp.where(mi, sf_i.astype(jnp.int32), spread) + N_NEI
        acc_v[r] = jnp.zeros((L,), jnp.float32)
        return carry

    lax.fori_loop(0, NV, scl_body, 0, unroll=2)

    eibufs = (ei0_v, ei1_v)
    dubufs = (du0_v, du1_v)
    dibufs = (di0_v, di1_v)
    sems = (semG0, semG1)

    def i_start(d, b):
        pltpu.async_copy(it_t.at[d].at[ti_v], eibufs[b], sems[b])
        pltpu.async_copy(p_t.at[d].at[cu_v], dubufs[b], sems[b])
        pltpu.async_copy(p_t.at[d].at[ci_v], dibufs[b], sems[b])

    def i_drain(b):
        pltpu.make_async_copy(it_t.at[0].at[ti_v], eibufs[b], sems[b]).wait()
        pltpu.make_async_copy(p_t.at[0].at[cu_v], dubufs[b], sems[b]).wait()
        pltpu.make_async_copy(p_t.at[0].at[ci_v], dibufs[b], sems[b]).wait()

    def i_fma(d, b):
        def fma_body(v, carry):
            r = pl.ds(v * L, L)
            bval = BUF[pl.ds(d * PAIRS_PER_W + v * L, L)]
            t1 = bval + su_v[r] * dubufs[b][r]
            t2 = eibufs[b][r] + si_v[r] * dibufs[b][r]
            acc_v[r] = acc_v[r] + t1 * t2
            return carry

        lax.fori_loop(0, NV, fma_body, 0, unroll=2)

    i_start(0, 0)

    def i_body(k, carry):
        d0 = 2 * k

        i_start(d0 + 1, 1)
        i_drain(0)
        i_fma(d0, 0)

        @pl.when(k < (D // 2 - 1))
        def _():
            i_start(d0 + 2, 0)

        i_drain(1)
        i_fma(d0 + 1, 1)
        return carry

    lax.fori_loop(0, D // 2, i_body, 0)

    pltpu.sync_copy(acc_v, scores_ref.at[pl.ds(pair_base, PAIRS_PER_W)])


def kernel(user_table, item_table, p, nei_users, nei_items,
           train_users, train_items):
    ut_t = user_table.T
    it_t = item_table.T
    p_t = p.reshape(N_PROWS, D).T
    scores, _ = _score_all(ut_t, it_t, p_t, nei_users, nei_items,
                           train_users, train_items)
    return scores


# speedup vs baseline: 5.3043x; 5.3043x over previous
import functools

import jax
import jax.numpy as jnp
from jax import lax
from jax.experimental import pallas as pl
from jax.experimental.pallas import tpu as pltpu
from jax.experimental.pallas import tpu_sc as plsc

D = 32
N_NEI = 16384
N_PAIRS = 65536
N_TRAIN = 65536
SCALE = float(2.0 ** -16)

NC = 2
NS = 16
NW = NC * NS
L = 16

MAP_SIZE = 1 << 20
ROWS_PER_W = MAP_SIZE // NW

N_PROWS = 2 * N_NEI
N_DUMMY = 1024
PAIRS_PER_W = N_PAIRS // NW
CH = 512
NCHUNK = PAIRS_PER_W // CH
QN = CH // 128

_mesh = plsc.VectorSubcoreMesh(core_axis_name="c", subcore_axis_name="s")

N_ROWS = 1000000
TBLK = 4096
NFULL = N_ROWS // TBLK
TAIL = N_ROWS - NFULL * TBLK


@functools.partial(
    pl.pallas_call,
    in_specs=[pl.BlockSpec(memory_space=pl.ANY)],
    out_specs=pl.BlockSpec(memory_space=pl.ANY),
    out_shape=jax.ShapeDtypeStruct((N_ROWS, D), jnp.float32),
    scratch_shapes=[
        pltpu.VMEM((D, TBLK), jnp.float32),
        pltpu.VMEM((D, TBLK), jnp.float32),
        pltpu.VMEM((TBLK, D), jnp.float32),
        pltpu.VMEM((TBLK, D), jnp.float32),
        pltpu.VMEM((D, TAIL), jnp.float32),
        pltpu.VMEM((TAIL, D), jnp.float32),
        pltpu.SemaphoreType.DMA,
        pltpu.SemaphoreType.DMA,
        pltpu.SemaphoreType.DMA,
        pltpu.SemaphoreType.DMA,
        pltpu.SemaphoreType.DMA,
        pltpu.SemaphoreType.DMA,
    ],
)
def _repack_table(x_hbm, o_hbm, vin0, vin1, vout0, vout1, vin_t, vout_t,
                  si0, si1, so0, so1, si_t, so_t):
    eye = jnp.eye(D, dtype=jnp.float32)

    def _tr(x):
        return lax.dot_general(x, eye, (((0,), (0,)), ((), ())),
                               preferred_element_type=jnp.float32)

    vins = (vin0, vin1)
    vouts = (vout0, vout1)
    sis = (si0, si1)
    sos = (so0, so1)

    def off_of(c):
        if isinstance(c, int):
            return c * TBLK
        return pl.multiple_of(c * TBLK, TBLK)

    def start_in(c, b):
        pltpu.make_async_copy(
            x_hbm.at[:, pl.ds(off_of(c), TBLK)], vins[b], sis[b]
        ).start()

    def wait_in(b):
        pltpu.make_async_copy(
            x_hbm.at[:, pl.ds(0, TBLK)], vins[b], sis[b]
        ).wait()

    def start_out(c, b):
        pltpu.make_async_copy(
            vouts[b], o_hbm.at[pl.ds(off_of(c), TBLK), :], sos[b]
        ).start()

    def wait_out(b):
        pltpu.make_async_copy(
            vouts[b], o_hbm.at[pl.ds(0, TBLK), :], sos[b]
        ).wait()

    start_in(0, 0)
    start_in(1, 1)
    tail_in = pltpu.make_async_copy(
        x_hbm.at[:, pl.ds(NFULL * TBLK, TAIL)], vin_t, si_t)
    tail_in.start()

    for b in range(2):
        wait_in(b)
        vouts[b][...] = _tr(vins[b][...])
        start_out(b, b)
        start_in(b + 2, b)

    def pair_body(k, carry):
        for b in range(2):
            c = k * 2 + b
            wait_in(b)
            wait_out(b)
            vouts[b][...] = _tr(vins[b][...])
            start_out(c, b)

            @pl.when(k < NFULL // 2 - 1)
            def _():
                start_in(c + 2, b)
        return carry

    lax.fori_loop(1, NFULL // 2, pair_body, 0)

    tail_in.wait()
    vout_t[...] = _tr(vin_t[...])
    tail_out = pltpu.make_async_copy(
        vout_t, o_hbm.at[pl.ds(NFULL * TBLK, TAIL), :], so_t)
    tail_out.start()
    wait_out(0)
    wait_out(1)
    tail_out.wait()


@functools.partial(
    pl.kernel,
    out_type=(
        jax.ShapeDtypeStruct((MAP_SIZE,), jnp.int32),
        jax.ShapeDtypeStruct((MAP_SIZE,), jnp.int32),
    ),
    mesh=_mesh,
    scratch_types=[
        pltpu.VMEM((ROWS_PER_W,), jnp.int32),
        pltpu.VMEM((ROWS_PER_W,), jnp.int32),
        pltpu.VMEM((N_NEI,), jnp.int32),
        pltpu.VMEM((N_NEI,), jnp.int32),
        pltpu.VMEM((L,), jnp.int32),
    ],
    compiler_params=pltpu.CompilerParams(needs_layout_passes=False, use_tc_tiling_on_sc=False),
)
def _build_maps(nei_u_hbm, nei_i_hbm, map_u_hbm, map_i_hbm,
                frag_u, frag_i, nei_u, nei_i, tmp16):
    wid = lax.axis_index("s") * NC + lax.axis_index("c")
    base = wid * ROWS_PER_W

    neg1 = jnp.full((L,), -1, jnp.int32)

    def init_body(i, carry):
        frag_u[pl.ds(i * L, L)] = neg1
        frag_i[pl.ds(i * L, L)] = neg1
        return carry

    lax.fori_loop(0, ROWS_PER_W // L, init_body, 0, unroll=4)

    pltpu.sync_copy(nei_u_hbm, nei_u)
    pltpu.sync_copy(nei_i_hbm, nei_i)

    lanes = lax.iota(jnp.int32, L)
    shift_idx = jnp.minimum(lanes + 1, L - 1)
    is_last_lane = lanes == (L - 1)

    def scatter_group(frag, nei_ref, g):
        idx = nei_ref[pl.ds(g * L, L)]
        key = (idx << 4) | lanes
        kvec = g * L + lanes
        skey, sval = plsc.sort_key_val(key, kvec)
        sidx = skey >> 4
        tmp16[...] = sidx
        nxt = plsc.load_gather(tmp16, [shift_idx])
        winner = (sidx != nxt) | is_last_lane
        local = sidx - base
        in_range = plsc.bitcast(local, jnp.uint32) < jnp.uint32(ROWS_PER_W)
        local_c = local & (ROWS_PER_W - 1)
        plsc.store_scatter(frag, [local_c], sval, mask=winner & in_range)

    def body(g, carry):
        scatter_group(frag_u, nei_u, g)
        scatter_group(frag_i, nei_i, g)
        return carry

    lax.fori_loop(0, N_NEI // L, body, 0)

    pltpu.sync_copy(frag_u, map_u_hbm.at[pl.ds(base, ROWS_PER_W)])
    pltpu.sync_copy(frag_i, map_i_hbm.at[pl.ds(base, ROWS_PER_W)])


@functools.partial(
    pl.kernel,
    out_type=jax.ShapeDtypeStruct((N_PAIRS,), jnp.float32),
    mesh=_mesh,
    scratch_types=[
        pltpu.VMEM((CH,), jnp.int32),
        pltpu.VMEM((CH,), jnp.int32),
        pltpu.VMEM((CH,), jnp.int32),
        pltpu.VMEM((CH,), jnp.int32),
        pltpu.VMEM((CH,), jnp.int32),
        pltpu.VMEM((CH,), jnp.int32),
        pltpu.VMEM((CH,), jnp.float32),
        pltpu.VMEM((CH,), jnp.float32),
        pltpu.VMEM((CH, D), jnp.float32),
        pltpu.VMEM((CH, D), jnp.float32),
        pltpu.VMEM((CH, D), jnp.float32),
        pltpu.VMEM((CH, D), jnp.float32),
        pltpu.VMEM((CH,), jnp.float32),
        pltpu.SemaphoreType.DMA,
        pltpu.SemaphoreType.DMA,
    ],
    compiler_params=pltpu.CompilerParams(needs_layout_passes=False, use_tc_tiling_on_sc=False),
)
def _scores(ut_hbm, it_hbm, pext_hbm, map_u_hbm, map_i_hbm,
            tu_hbm, ti_hbm, out_hbm,
            tu, ti, slot_u, slot_i, pidx_u, pidx_i, scale_u, scale_i,
            gu, gi, pu, pi, sbuf, sem_a, sem_b):
    wid = lax.axis_index("s") * NC + lax.axis_index("c")
    lanes = lax.iota(jnp.int32, L)
    scale = jnp.float32(SCALE)
    zero = jnp.float32(0.0)

    def chunk_body(c, carry):
        pair_base = wid * PAIRS_PER_W + c * CH
        pltpu.sync_copy(tu_hbm.at[pl.ds(pair_base, CH)], tu)
        pltpu.sync_copy(ti_hbm.at[pl.ds(pair_base, CH)], ti)
        for q in range(QN):
            iu = tu.at[pl.ds(q * 128, 128)]
            ii = ti.at[pl.ds(q * 128, 128)]
            pltpu.async_copy(map_u_hbm.at[iu], slot_u.at[pl.ds(q * 128, 128)], sem_a)
            pltpu.async_copy(map_i_hbm.at[ii], slot_i.at[pl.ds(q * 128, 128)], sem_a)
            pltpu.async_copy(ut_hbm.at[iu], gu.at[pl.ds(q * 128, 128), :], sem_b)
            pltpu.async_copy(it_hbm.at[ii], gi.at[pl.ds(q * 128, 128), :], sem_b)
        for q in range(QN):
            iu = tu.at[pl.ds(q * 128, 128)]
            ii = ti.at[pl.ds(q * 128, 128)]
            pltpu.make_async_copy(map_u_hbm.at[iu], slot_u.at[pl.ds(q * 128, 128)], sem_a).wait()
            pltpu.make_async_copy(map_i_hbm.at[ii], slot_i.at[pl.ds(q * 128, 128)], sem_a).wait()

        def pidx_body(g, carry2):
            su = slot_u[pl.ds(g * L, L)]
            si = slot_i[pl.ds(g * L, L)]
            mu = su >= 0
            mi = si >= 0
            spread = (pair_base + g * L + lanes) & (N_NEI - 1)
            pidx_u[pl.ds(g * L, L)] = jnp.where(mu, su, spread)
            pidx_i[pl.ds(g * L, L)] = jnp.where(mi, si, spread) + N_NEI
            scale_u[pl.ds(g * L, L)] = jnp.where(mu, scale, zero)
            scale_i[pl.ds(g * L, L)] = jnp.where(mi, scale, zero)
            return carry2

        lax.fori_loop(0, CH // L, pidx_body, 0, unroll=4)

        for q in range(QN):
            pltpu.async_copy(pext_hbm.at[pidx_u.at[pl.ds(q * 128, 128)]],
                             pu.at[pl.ds(q * 128, 128), :], sem_a)
            pltpu.async_copy(pext_hbm.at[pidx_i.at[pl.ds(q * 128, 128)]],
                             pi.at[pl.ds(q * 128, 128), :], sem_a)
        for q in range(QN):
            pltpu.make_async_copy(ut_hbm.at[tu.at[pl.ds(q * 128, 128)]],
                                  gu.at[pl.ds(q * 128, 128), :], sem_b).wait()
            pltpu.make_async_copy(it_hbm.at[ti.at[pl.ds(q * 128, 128)]],
                                  gi.at[pl.ds(q * 128, 128), :], sem_b).wait()
            pltpu.make_async_copy(pext_hbm.at[pidx_u.at[pl.ds(q * 128, 128)]],
                                  pu.at[pl.ds(q * 128, 128), :], sem_a).wait()
            pltpu.make_async_copy(pext_hbm.at[pidx_i.at[pl.ds(q * 128, 128)]],
                                  pi.at[pl.ds(q * 128, 128), :], sem_a).wait()

        def dot_body(grp, carry2):
            rows = grp * L + lanes
            scu = scale_u[pl.ds(grp * L, L)]
            sci = scale_i[pl.ds(grp * L, L)]
            acc = jnp.zeros((L,), jnp.float32)
            for d in range(D):
                cold = jnp.full((L,), d, jnp.int32)
                au = plsc.load_gather(gu, [rows, cold])
                du = plsc.load_gather(pu, [rows, cold])
                ai = plsc.load_gather(gi, [rows, cold])
                di = plsc.load_gather(pi, [rows, cold])
                acc = acc + (au + scu * du) * (ai + sci * di)
            sbuf[pl.ds(grp * L, L)] = acc
            return carry2

        lax.fori_loop(0, CH // L, dot_body, 0)
        pltpu.sync_copy(sbuf, out_hbm.at[pl.ds(pair_base, CH)])
        return carry

    lax.fori_loop(0, NCHUNK, chunk_body, 0)


def kernel(user_table, item_table, p, nei_users, nei_items,
           train_users, train_items):
    map_u, map_i = _build_maps(nei_users, nei_items)
    return _scores(user_table, item_table, p.reshape(N_PROWS, D),
                   map_u, map_i, train_users, train_items)


# speedup vs baseline: 5.3082x; 1.0007x over previous
import functools

import jax
import jax.numpy as jnp
from jax import lax
from jax.experimental import pallas as pl
from jax.experimental.pallas import tpu as pltpu
from jax.experimental.pallas import tpu_sc as plsc

D = 32
N_NEI = 16384
N_PAIRS = 65536
N_TRAIN = 65536
SCALE = float(2.0 ** -16)

NC = 2
NS = 16
NW = NC * NS
L = 16

MAP_SIZE = 1 << 20
ROWS_PER_W = MAP_SIZE // NW

N_PROWS = 2 * N_NEI
N_DUMMY = 1024
PAIRS_PER_W = N_PAIRS // NW
CH = 512
NCHUNK = PAIRS_PER_W // CH
QN = CH // 128

_mesh = plsc.VectorSubcoreMesh(core_axis_name="c", subcore_axis_name="s")

@functools.partial(
    pl.kernel,
    out_type=(
        jax.ShapeDtypeStruct((MAP_SIZE,), jnp.int32),
        jax.ShapeDtypeStruct((MAP_SIZE,), jnp.int32),
    ),
    mesh=_mesh,
    scratch_types=[
        pltpu.VMEM((ROWS_PER_W,), jnp.int32),
        pltpu.VMEM((ROWS_PER_W,), jnp.int32),
        pltpu.VMEM((N_NEI,), jnp.int32),
        pltpu.VMEM((N_NEI,), jnp.int32),
        pltpu.VMEM((L,), jnp.int32),
    ],
    compiler_params=pltpu.CompilerParams(needs_layout_passes=False, use_tc_tiling_on_sc=False),
)
def _build_maps(nei_u_hbm, nei_i_hbm, map_u_hbm, map_i_hbm,
                frag_u, frag_i, nei_u, nei_i, tmp16):
    wid = lax.axis_index("s") * NC + lax.axis_index("c")
    base = wid * ROWS_PER_W

    neg1 = jnp.full((L,), -1, jnp.int32)

    def init_body(i, carry):
        frag_u[pl.ds(i * L, L)] = neg1
        frag_i[pl.ds(i * L, L)] = neg1
        return carry

    lax.fori_loop(0, ROWS_PER_W // L, init_body, 0, unroll=4)

    pltpu.sync_copy(nei_u_hbm, nei_u)
    pltpu.sync_copy(nei_i_hbm, nei_i)

    lanes = lax.iota(jnp.int32, L)
    shift_idx = jnp.minimum(lanes + 1, L - 1)
    is_last_lane = lanes == (L - 1)

    def scatter_group(frag, nei_ref, g):
        idx = nei_ref[pl.ds(g * L, L)]
        key = (idx << 4) | lanes
        kvec = g * L + lanes
        skey, sval = plsc.sort_key_val(key, kvec)
        sidx = skey >> 4
        tmp16[...] = sidx
        nxt = plsc.load_gather(tmp16, [shift_idx])
        winner = (sidx != nxt) | is_last_lane
        local = sidx - base
        in_range = plsc.bitcast(local, jnp.uint32) < jnp.uint32(ROWS_PER_W)
        local_c = local & (ROWS_PER_W - 1)
        plsc.store_scatter(frag, [local_c], sval, mask=winner & in_range)

    def body(g, carry):
        scatter_group(frag_u, nei_u, g)
        scatter_group(frag_i, nei_i, g)
        return carry

    lax.fori_loop(0, N_NEI // L, body, 0)

    pltpu.sync_copy(frag_u, map_u_hbm.at[pl.ds(base, ROWS_PER_W)])
    pltpu.sync_copy(frag_i, map_i_hbm.at[pl.ds(base, ROWS_PER_W)])


@functools.partial(
    pl.kernel,
    out_type=jax.ShapeDtypeStruct((N_PAIRS,), jnp.float32),
    mesh=_mesh,
    scratch_types=[
        pltpu.VMEM((CH,), jnp.int32),
        pltpu.VMEM((CH,), jnp.int32),
        pltpu.VMEM((CH,), jnp.int32),
        pltpu.VMEM((CH,), jnp.int32),
        pltpu.VMEM((CH,), jnp.int32),
        pltpu.VMEM((CH,), jnp.int32),
        pltpu.VMEM((CH,), jnp.float32),
        pltpu.VMEM((CH,), jnp.float32),
        pltpu.VMEM((CH, D), jnp.float32),
        pltpu.VMEM((CH, D), jnp.float32),
        pltpu.VMEM((CH, D), jnp.float32),
        pltpu.VMEM((CH, D), jnp.float32),
        pltpu.VMEM((CH,), jnp.float32),
        pltpu.SemaphoreType.DMA,
        pltpu.SemaphoreType.DMA,
    ],
    compiler_params=pltpu.CompilerParams(needs_layout_passes=False, use_tc_tiling_on_sc=False),
)
def _scores(ut_hbm, it_hbm, pext_hbm, map_u_hbm, map_i_hbm,
            tu_hbm, ti_hbm, out_hbm,
            tu, ti, slot_u, slot_i, pidx_u, pidx_i, scale_u, scale_i,
            gu, gi, pu, pi, sbuf, sem_a, sem_b):
    wid = lax.axis_index("s") * NC + lax.axis_index("c")
    lanes = lax.iota(jnp.int32, L)
    scale = jnp.float32(SCALE)
    zero = jnp.float32(0.0)

    def chunk_body(c, carry):
        pair_base = wid * PAIRS_PER_W + c * CH
        pltpu.sync_copy(tu_hbm.at[pl.ds(pair_base, CH)], tu)
        pltpu.sync_copy(ti_hbm.at[pl.ds(pair_base, CH)], ti)
        for q in range(QN):
            iu = tu.at[pl.ds(q * 128, 128)]
            ii = ti.at[pl.ds(q * 128, 128)]
            pltpu.async_copy(map_u_hbm.at[iu], slot_u.at[pl.ds(q * 128, 128)], sem_a)
            pltpu.async_copy(map_i_hbm.at[ii], slot_i.at[pl.ds(q * 128, 128)], sem_a)
            pltpu.async_copy(ut_hbm.at[iu], gu.at[pl.ds(q * 128, 128), :], sem_b)
            pltpu.async_copy(it_hbm.at[ii], gi.at[pl.ds(q * 128, 128), :], sem_b)
        for q in range(QN):
            iu = tu.at[pl.ds(q * 128, 128)]
            ii = ti.at[pl.ds(q * 128, 128)]
            pltpu.make_async_copy(map_u_hbm.at[iu], slot_u.at[pl.ds(q * 128, 128)], sem_a).wait()
            pltpu.make_async_copy(map_i_hbm.at[ii], slot_i.at[pl.ds(q * 128, 128)], sem_a).wait()

        def pidx_body(g, carry2):
            su = slot_u[pl.ds(g * L, L)]
            si = slot_i[pl.ds(g * L, L)]
            mu = su >= 0
            mi = si >= 0
            spread = (pair_base + g * L + lanes) & (N_NEI - 1)
            pidx_u[pl.ds(g * L, L)] = jnp.where(mu, su, spread)
            pidx_i[pl.ds(g * L, L)] = jnp.where(mi, si, spread) + N_NEI
            scale_u[pl.ds(g * L, L)] = jnp.where(mu, scale, zero)
            scale_i[pl.ds(g * L, L)] = jnp.where(mi, scale, zero)
            return carry2

        lax.fori_loop(0, CH // L, pidx_body, 0, unroll=4)

        for q in range(QN):
            pltpu.async_copy(pext_hbm.at[pidx_u.at[pl.ds(q * 128, 128)]],
                             pu.at[pl.ds(q * 128, 128), :], sem_a)
            pltpu.async_copy(pext_hbm.at[pidx_i.at[pl.ds(q * 128, 128)]],
                             pi.at[pl.ds(q * 128, 128), :], sem_a)
        for q in range(QN):
            pltpu.make_async_copy(ut_hbm.at[tu.at[pl.ds(q * 128, 128)]],
                                  gu.at[pl.ds(q * 128, 128), :], sem_b).wait()
            pltpu.make_async_copy(it_hbm.at[ti.at[pl.ds(q * 128, 128)]],
                                  gi.at[pl.ds(q * 128, 128), :], sem_b).wait()
            pltpu.make_async_copy(pext_hbm.at[pidx_u.at[pl.ds(q * 128, 128)]],
                                  pu.at[pl.ds(q * 128, 128), :], sem_a).wait()
            pltpu.make_async_copy(pext_hbm.at[pidx_i.at[pl.ds(q * 128, 128)]],
                                  pi.at[pl.ds(q * 128, 128), :], sem_a).wait()

        def dot_body(grp, carry2):
            rows = grp * L + lanes
            scu = scale_u[pl.ds(grp * L, L)]
            sci = scale_i[pl.ds(grp * L, L)]
            acc = jnp.zeros((L,), jnp.float32)
            for d in range(D):
                cold = jnp.full((L,), d, jnp.int32)
                au = plsc.load_gather(gu, [rows, cold])
                du = plsc.load_gather(pu, [rows, cold])
                ai = plsc.load_gather(gi, [rows, cold])
                di = plsc.load_gather(pi, [rows, cold])
                acc = acc + (au + scu * du) * (ai + sci * di)
            sbuf[pl.ds(grp * L, L)] = acc
            return carry2

        lax.fori_loop(0, CH // L, dot_body, 0)
        pltpu.sync_copy(sbuf, out_hbm.at[pl.ds(pair_base, CH)])
        return carry

    lax.fori_loop(0, NCHUNK, chunk_body, 0)


def kernel(user_table, item_table, p, nei_users, nei_items,
           train_users, train_items):
    map_u, map_i = _build_maps(nei_users, nei_items)
    return _scores(user_table, item_table, p.reshape(N_PROWS, D),
                   map_u, map_i, train_users, train_items)
